# baseline (speedup 1.0000x reference)
import functools

import jax
import jax.numpy as jnp
from jax import lax
from jax.experimental import pallas as pl
from jax.experimental.pallas import tpu as pltpu
from jax.experimental.pallas import tpu_sc as plsc

N = 100000
E = 6400000
INV_T = 100.0

NC, NS, L = 2, 16, 16
NW = NC * NS
NPAD = 100352
NSLICE = NPAD // NS
TP = NPAD // 16
CROWS = 16
NBLOCKS = E // 128 // CROWS


def _sc_body(x_hbm, tp_hbm, src_hbm, dst_hbm, z_hbm, nump_hbm, denp_hbm,
             x_v, tp_v, src_v, dst_v, num_v, den_v, num_sh, den_sh, sem):
    cid = lax.axis_index("c")
    sid = lax.axis_index("s")
    wid = sid * NC + cid

    pltpu.sync_copy(x_hbm, x_v)
    pltpu.sync_copy(tp_hbm, tp_v)

    nbase = sid * NSLICE
    pltpu.sync_copy(z_hbm, num_sh.at[pl.ds(nbase, NSLICE)])
    pltpu.sync_copy(z_hbm, den_sh.at[pl.ds(nbase, NSLICE)])
    plsc.subcore_barrier()

    trips = jnp.where(wid < NBLOCKS - (NBLOCKS // NW) * NW, NBLOCKS // NW + 1,
                      NBLOCKS // NW)

    def chunk_body(c, carry):
        r = (wid + NW * c) * CROWS
        pltpu.sync_copy(src_hbm.at[pl.ds(r, CROWS)], src_v)
        pltpu.sync_copy(dst_hbm.at[pl.ds(r, CROWS)], dst_v)
        descs = []
        for j in range(CROWS):
            srow = src_v.at[j]
            drow = dst_v.at[j]
            nrow = num_v.at[j]
            wrow = den_v.at[j]

            def lane_body(i, c2):
                s = srow[pl.ds(i * L, L)]
                d = drow[pl.ds(i * L, L)]
                v = plsc.load_gather(x_v, [s])
                word = plsc.load_gather(tp_v, [lax.shift_right_logical(d, 4)])
                t = lax.shift_right_logical(word, 2 * (d & 15)) & 3
                a = v * INV_T
                arg = jnp.where(t == 1, -a, a - INV_T)
                wgt = jnp.where(arg < -87.0, 0.0, jnp.exp(arg))
                nrow[pl.ds(i * L, L)] = jnp.where(t == 3, 1.0 - v, wgt * v)
                wrow[pl.ds(i * L, L)] = wgt
                return c2

            lax.fori_loop(0, 128 // L, lane_body, 0)
            descs.append(pltpu.async_copy(nrow, num_sh.at[drow], sem,
                                          add=True))
            descs.append(pltpu.async_copy(wrow, den_sh.at[drow], sem,
                                          add=True))
        for dsc in descs:
            dsc.wait()
        return carry

    lax.fori_loop(0, trips, chunk_body, 0)
    plsc.subcore_barrier()

    pltpu.sync_copy(num_sh.at[pl.ds(nbase, NSLICE)],
                    nump_hbm.at[cid, pl.ds(nbase, NSLICE)])
    pltpu.sync_copy(den_sh.at[pl.ds(nbase, NSLICE)],
                    denp_hbm.at[cid, pl.ds(nbase, NSLICE)])


_sc_accumulate = pl.kernel(
    _sc_body,
    out_type=(
        jax.ShapeDtypeStruct((NC, NPAD), jnp.float32),
        jax.ShapeDtypeStruct((NC, NPAD), jnp.float32),
    ),
    mesh=plsc.VectorSubcoreMesh(core_axis_name="c", subcore_axis_name="s"),
    compiler_params=pltpu.CompilerParams(needs_layout_passes=False),
    scratch_types=[
        pltpu.VMEM((N,), jnp.float32),
        pltpu.VMEM((TP,), jnp.int32),
        pltpu.VMEM((CROWS, 128), jnp.int32),
        pltpu.VMEM((CROWS, 128), jnp.int32),
        pltpu.VMEM((CROWS, 128), jnp.float32),
        pltpu.VMEM((CROWS, 128), jnp.float32),
        pltpu.VMEM_SHARED((NPAD,), jnp.float32),
        pltpu.VMEM_SHARED((NPAD,), jnp.float32),
        pltpu.SemaphoreType.DMA,
    ],
)


def _finalize_body(nump_ref, denp_ref, m12_ref, m3_ref, o_ref):
    num = nump_ref[0] + nump_ref[1]
    den = denp_ref[0] + denp_ref[1]
    o_ref[...] = jnp.where(m3_ref[...] > 0.0, num,
                           m12_ref[...] * (num / (den + 1e-30)))


_finalize = pl.pallas_call(
    _finalize_body,
    out_shape=jax.ShapeDtypeStruct((NPAD // 128, 128), jnp.float32),
)


@jax.jit
def kernel(x, node_attr, edge_index):
    xf = x[:, 0]

    t = jnp.argmax(node_attr, axis=1).astype(jnp.int32)
    t = jnp.concatenate([t, jnp.zeros((NPAD - N,), jnp.int32)])
    shifts = 2 * jnp.arange(16, dtype=jnp.int32)
    packed = jnp.sum(t.reshape(TP, 16) << shifts[None, :], axis=1,
                     dtype=jnp.int32)

    src2d = edge_index[0].reshape(E // 128, 128)
    dst2d = edge_index[1].reshape(E // 128, 128)

    z = jnp.zeros((NSLICE,), jnp.float32)
    nump, denp = _sc_accumulate(xf, packed, src2d, dst2d, z)

    m12 = jnp.concatenate([node_attr[:, 1] + node_attr[:, 2],
                           jnp.zeros((NPAD - N,), jnp.float32)])
    m3 = jnp.concatenate([node_attr[:, 3], jnp.zeros((NPAD - N,),
                                                     jnp.float32)])
    out = _finalize(nump.reshape(NC, NPAD // 128, 128),
                    denp.reshape(NC, NPAD // 128, 128),
                    m12.reshape(NPAD // 128, 128),
                    m3.reshape(NPAD // 128, 128))
    return out.reshape(NPAD)[:N][:, None]


# speedup vs baseline: 1.3705x; 1.3705x over previous
import functools

import jax
import jax.numpy as jnp
from jax import lax
from jax.experimental import pallas as pl
from jax.experimental.pallas import tpu as pltpu
from jax.experimental.pallas import tpu_sc as plsc

N = 100000
E = 6400000
INV_T = 100.0

NC, NS, L = 2, 16, 16
NW = NC * NS
NPAD = 100352
NSLICE = NPAD // NS
TP = NPAD // 16
CROWS = 16
NBLOCKS = E // 128 // CROWS


def _sc_body(x_hbm, tp_hbm, src_hbm, dst_hbm, z_hbm, nump_hbm, denp_hbm,
             x_v, tp_v, src_v, dst_v, num_v, den_v, num_sh, den_sh, sem):
    cid = lax.axis_index("c")
    sid = lax.axis_index("s")
    wid = sid * NC + cid

    pltpu.sync_copy(x_hbm, x_v)
    pltpu.sync_copy(tp_hbm, tp_v)

    nbase = sid * NSLICE
    pltpu.sync_copy(z_hbm, num_sh.at[pl.ds(nbase, NSLICE)])
    pltpu.sync_copy(z_hbm, den_sh.at[pl.ds(nbase, NSLICE)])
    plsc.subcore_barrier()

    trips = jnp.where(wid < NBLOCKS - (NBLOCKS // NW) * NW, NBLOCKS // NW + 1,
                      NBLOCKS // NW)

    def chunk_body(c, carry):
        r = (wid + NW * c) * CROWS
        pltpu.sync_copy(src_hbm.at[pl.ds(r, CROWS)], src_v)
        pltpu.sync_copy(dst_hbm.at[pl.ds(r, CROWS)], dst_v)
        descs = []
        for j in range(CROWS):
            srow = src_v.at[j]
            drow = dst_v.at[j]
            nrow = num_v.at[j]
            wrow = den_v.at[j]

            def lane_body(i, c2):
                s = srow[pl.ds(i * L, L)]
                d = drow[pl.ds(i * L, L)]
                v = plsc.load_gather(x_v, [s])
                word = plsc.load_gather(tp_v, [lax.shift_right_logical(d, 4)])
                t = lax.shift_right_logical(word, 2 * (d & 15)) & 3
                is1 = t == 1
                arg = jnp.where(is1, v * (-INV_T), v * INV_T - INV_T)
                wgt = jnp.where(is1 | (t == 2), jnp.exp(arg), 0.0)
                nrow[pl.ds(i * L, L)] = jnp.where(t == 3, 1.0 - v, wgt * v)
                wrow[pl.ds(i * L, L)] = wgt
                return c2

            lax.fori_loop(0, 128 // L, lane_body, 0)
            descs.append(pltpu.async_copy(nrow, num_sh.at[drow], sem,
                                          add=True))
            descs.append(pltpu.async_copy(wrow, den_sh.at[drow], sem,
                                          add=True))
        for dsc in descs:
            dsc.wait()
        return carry

    lax.fori_loop(0, trips, chunk_body, 0)
    plsc.subcore_barrier()

    pltpu.sync_copy(num_sh.at[pl.ds(nbase, NSLICE)],
                    nump_hbm.at[cid, pl.ds(nbase, NSLICE)])
    pltpu.sync_copy(den_sh.at[pl.ds(nbase, NSLICE)],
                    denp_hbm.at[cid, pl.ds(nbase, NSLICE)])


_sc_accumulate = pl.kernel(
    _sc_body,
    out_type=(
        jax.ShapeDtypeStruct((NC, NPAD), jnp.float32),
        jax.ShapeDtypeStruct((NC, NPAD), jnp.float32),
    ),
    mesh=plsc.VectorSubcoreMesh(core_axis_name="c", subcore_axis_name="s"),
    compiler_params=pltpu.CompilerParams(needs_layout_passes=False),
    scratch_types=[
        pltpu.VMEM((N,), jnp.float32),
        pltpu.VMEM((TP,), jnp.int32),
        pltpu.VMEM((CROWS, 128), jnp.int32),
        pltpu.VMEM((CROWS, 128), jnp.int32),
        pltpu.VMEM((CROWS, 128), jnp.float32),
        pltpu.VMEM((CROWS, 128), jnp.float32),
        pltpu.VMEM_SHARED((NPAD,), jnp.float32),
        pltpu.VMEM_SHARED((NPAD,), jnp.float32),
        pltpu.SemaphoreType.DMA,
    ],
)


def _finalize_body(nump_ref, denp_ref, m12_ref, m3_ref, o_ref):
    num = nump_ref[0] + nump_ref[1]
    den = denp_ref[0] + denp_ref[1]
    o_ref[...] = jnp.where(m3_ref[...] > 0.0, num,
                           m12_ref[...] * (num / (den + 1e-30)))


_finalize = pl.pallas_call(
    _finalize_body,
    out_shape=jax.ShapeDtypeStruct((NPAD // 128, 128), jnp.float32),
)


@jax.jit
def kernel(x, node_attr, edge_index):
    xf = x[:, 0]

    t = jnp.argmax(node_attr, axis=1).astype(jnp.int32)
    t = jnp.concatenate([t, jnp.zeros((NPAD - N,), jnp.int32)])
    shifts = 2 * jnp.arange(16, dtype=jnp.int32)
    packed = jnp.sum(t.reshape(TP, 16) << shifts[None, :], axis=1,
                     dtype=jnp.int32)

    src2d = edge_index[0].reshape(E // 128, 128)
    dst2d = edge_index[1].reshape(E // 128, 128)

    z = jnp.zeros((NSLICE,), jnp.float32)
    nump, denp = _sc_accumulate(xf, packed, src2d, dst2d, z)

    m12 = jnp.concatenate([node_attr[:, 1] + node_attr[:, 2],
                           jnp.zeros((NPAD - N,), jnp.float32)])
    m3 = jnp.concatenate([node_attr[:, 3], jnp.zeros((NPAD - N,),
                                                     jnp.float32)])
    out = _finalize(nump.reshape(NC, NPAD // 128, 128),
                    denp.reshape(NC, NPAD // 128, 128),
                    m12.reshape(NPAD // 128, 128),
                    m3.reshape(NPAD // 128, 128))
    return out.reshape(NPAD)[:N][:, None]


# speedup vs baseline: 1.5543x; 1.1341x over previous
import functools

import jax
import jax.numpy as jnp
from jax import lax
from jax.experimental import pallas as pl
from jax.experimental.pallas import tpu as pltpu
from jax.experimental.pallas import tpu_sc as plsc

N = 100000
E = 6400000
INV_T = 100.0

NC, NS, L = 2, 16, 16
NW = NC * NS
NPAD = 100096
NSLICE = 6272
NSLICE_LAST = NPAD - 15 * NSLICE
TP = 6250
CROWS = 8
NBLOCKS = E // 128 // CROWS
NBUF = 3


def _sc_body(x_hbm, tp_hbm, src_hbm, dst_hbm, z_hbm, nump_hbm, denp_hbm,
             x_v, tp_v, src_v, dst_v, num_v, den_v, num_sh, den_sh, sem,
             lsem):
    cid = lax.axis_index("c")
    sid = lax.axis_index("s")
    wid = sid * NC + cid

    pltpu.sync_copy(x_hbm, x_v)
    pltpu.sync_copy(tp_hbm, tp_v)

    nbase = sid * NSLICE

    @pl.when(sid < NS - 1)
    def _():
        pltpu.sync_copy(z_hbm, num_sh.at[pl.ds(nbase, NSLICE)])
        pltpu.sync_copy(z_hbm, den_sh.at[pl.ds(nbase, NSLICE)])

    @pl.when(sid == NS - 1)
    def _():
        pltpu.sync_copy(z_hbm.at[pl.ds(0, NSLICE_LAST)],
                        num_sh.at[pl.ds(nbase, NSLICE_LAST)])
        pltpu.sync_copy(z_hbm.at[pl.ds(0, NSLICE_LAST)],
                        den_sh.at[pl.ds(nbase, NSLICE_LAST)])

    plsc.subcore_barrier()

    trips = jnp.where(wid < NBLOCKS - (NBLOCKS // NW) * NW, NBLOCKS // NW + 1,
                      NBLOCKS // NW)

    def row_of(c):
        return (wid + NW * c) * CROWS

    def start_loads(c, p):
        pltpu.async_copy(src_hbm.at[pl.ds(row_of(c), CROWS)],
                         src_v.at[c % 2], lsem)
        pltpu.async_copy(dst_hbm.at[pl.ds(row_of(c), CROWS)], dst_v.at[p],
                         lsem)

    def wait_loads(c, p):
        pltpu.make_async_copy(src_hbm.at[pl.ds(row_of(c), CROWS)],
                              src_v.at[c % 2], lsem).wait()
        pltpu.make_async_copy(dst_hbm.at[pl.ds(row_of(c), CROWS)],
                              dst_v.at[p], lsem).wait()

    def drain_scatters(p):
        for j in range(CROWS):
            pltpu.make_async_copy(num_v.at[p].at[j],
                                  num_sh.at[dst_v.at[p].at[j]], sem).wait()
            pltpu.make_async_copy(den_v.at[p].at[j],
                                  den_sh.at[dst_v.at[p].at[j]], sem).wait()

    start_loads(0, 0)

    def chunk_body(c, carry):
        p = c % NBUF
        q = (c + 1) % NBUF

        @pl.when(c + 1 >= NBUF)
        def _():
            drain_scatters(q)

        @pl.when(c + 1 < trips)
        def _():
            start_loads(c + 1, q)

        wait_loads(c, p)
        for j in range(CROWS):
            srow = src_v.at[c % 2].at[j]
            drow = dst_v.at[p].at[j]
            nrow = num_v.at[p].at[j]
            wrow = den_v.at[p].at[j]

            def lane_body(i, c2):
                s = srow[pl.ds(i * L, L)]
                d = drow[pl.ds(i * L, L)]
                v = plsc.load_gather(x_v, [s])
                word = plsc.load_gather(tp_v, [lax.shift_right_logical(d, 4)])
                t = lax.shift_right_logical(word, 2 * (d & 15)) & 3
                is1 = t == 1
                arg = jnp.where(is1, v * (-INV_T), v * INV_T - INV_T)
                wgt = jnp.where(is1 | (t == 2), jnp.exp(arg), 0.0)
                nrow[pl.ds(i * L, L)] = jnp.where(t == 3, 1.0 - v, wgt * v)
                wrow[pl.ds(i * L, L)] = wgt
                return c2

            lax.fori_loop(0, 128 // L, lane_body, 0)
            pltpu.async_copy(nrow, num_sh.at[drow], sem, add=True)
            pltpu.async_copy(wrow, den_sh.at[drow], sem, add=True)
        return carry

    lax.fori_loop(0, trips, chunk_body, 0)

    def ep_body(k, carry):
        drain_scatters(k % NBUF)
        return carry

    lax.fori_loop(trips - (NBUF - 1), trips, ep_body, 0)
    plsc.subcore_barrier()

    @pl.when(sid < NS - 1)
    def _():
        pltpu.sync_copy(num_sh.at[pl.ds(nbase, NSLICE)],
                        nump_hbm.at[pl.ds(cid * NPAD + nbase, NSLICE)])
        pltpu.sync_copy(den_sh.at[pl.ds(nbase, NSLICE)],
                        denp_hbm.at[pl.ds(cid * NPAD + nbase, NSLICE)])

    @pl.when(sid == NS - 1)
    def _():
        pltpu.sync_copy(num_sh.at[pl.ds(nbase, NSLICE_LAST)],
                        nump_hbm.at[pl.ds(cid * NPAD + nbase, NSLICE_LAST)])
        pltpu.sync_copy(den_sh.at[pl.ds(nbase, NSLICE_LAST)],
                        denp_hbm.at[pl.ds(cid * NPAD + nbase, NSLICE_LAST)])


_sc_accumulate = pl.kernel(
    _sc_body,
    out_type=(
        jax.ShapeDtypeStruct((NC * NPAD,), jnp.float32),
        jax.ShapeDtypeStruct((NC * NPAD,), jnp.float32),
    ),
    mesh=plsc.VectorSubcoreMesh(core_axis_name="c", subcore_axis_name="s"),
    compiler_params=pltpu.CompilerParams(needs_layout_passes=False),
    scratch_types=[
        pltpu.VMEM((N,), jnp.float32),
        pltpu.VMEM((TP,), jnp.int32),
        pltpu.VMEM((2, CROWS, 128), jnp.int32),
        pltpu.VMEM((NBUF, CROWS, 128), jnp.int32),
        pltpu.VMEM((NBUF, CROWS, 128), jnp.float32),
        pltpu.VMEM((NBUF, CROWS, 128), jnp.float32),
        pltpu.VMEM_SHARED((NPAD,), jnp.float32),
        pltpu.VMEM_SHARED((NPAD,), jnp.float32),
        pltpu.SemaphoreType.DMA,
        pltpu.SemaphoreType.DMA,
    ],
)


def _finalize_body(nump_ref, denp_ref, m12_ref, m3_ref, o_ref):
    num = nump_ref[0] + nump_ref[1]
    den = denp_ref[0] + denp_ref[1]
    o_ref[...] = jnp.where(m3_ref[...] > 0.0, num,
                           m12_ref[...] * (num / (den + 1e-30)))


_finalize = pl.pallas_call(
    _finalize_body,
    out_shape=jax.ShapeDtypeStruct((NPAD // 128, 128), jnp.float32),
)


@jax.jit
def kernel(x, node_attr, edge_index):
    xf = x[:, 0]

    t = jnp.argmax(node_attr, axis=1).astype(jnp.int32)
    shifts = 2 * jnp.arange(16, dtype=jnp.int32)
    packed = jnp.sum(t.reshape(TP, 16) << shifts[None, :], axis=1,
                     dtype=jnp.int32)

    src2d = edge_index[0].reshape(E // 128, 128)
    dst2d = edge_index[1].reshape(E // 128, 128)

    z = jnp.zeros((NSLICE,), jnp.float32)
    nump, denp = _sc_accumulate(xf, packed, src2d, dst2d, z)

    m12 = jnp.concatenate([node_attr[:, 1] + node_attr[:, 2],
                           jnp.zeros((NPAD - N,), jnp.float32)])
    m3 = jnp.concatenate([node_attr[:, 3], jnp.zeros((NPAD - N,),
                                                     jnp.float32)])
    out = _finalize(nump.reshape(NC, NPAD // 128, 128),
                    denp.reshape(NC, NPAD // 128, 128),
                    m12.reshape(NPAD // 128, 128),
                    m3.reshape(NPAD // 128, 128))
    return out.reshape(NPAD)[:N][:, None]
